# Initial kernel scaffold; baseline (speedup 1.0000x reference)
import jax
import jax.numpy as jnp
from jax.experimental import pallas as pl


def kernel(x, edge_index, W_l0, b_l0, W_r0, W_l1, b_l1, W_r1):
    raise NotImplementedError("write your pallas kernel here")


# speedup vs baseline: 2.9153x; 2.9153x over previous
import functools

import jax
import jax.numpy as jnp
from jax import lax
from jax.experimental import pallas as pl
from jax.experimental.pallas import tpu as pltpu
from jax.experimental.pallas import tpu_sc as plsc

N = 10000
E = 320000
D = 128

NC = 2
NS = 16
NT = NC * NS
CHUNK = 128
CPT = 80
E_PAD = NT * CPT * CHUNK
N_ACC = 10240

ROWS_PER_TILE = N_ACC // NS


def _sc_agg_body(with_deg, y_hbm, src_hbm, dst_hbm, psum_hbm, *rest):
    if with_deg:
        deg_hbm, src_v, dst_v, rows_v, deg_v, psum_acc_ref, sem = rest
    else:
        src_v, dst_v, rows_v, deg_v, psum_acc_ref, sem = rest
    c = lax.axis_index("c")
    s = lax.axis_index("s")
    wid = c * NS + s

    zero16 = jnp.zeros((16,), jnp.float32)

    def _zrow(r, _):
        for l in range(D // 16):
            rows_v[r, pl.ds(l * 16, 16)] = zero16
        return 0
    lax.fori_loop(0, CHUNK, _zrow, 0)

    if with_deg:
        def _zdeg(i, _):
            deg_v[pl.ds(i * 16, 16)] = zero16
            return 0
        lax.fori_loop(0, N_ACC // 16, _zdeg, 0)

    pltpu.sync_copy(src_hbm.at[pl.ds(wid * CPT, CPT)], src_v)
    pltpu.sync_copy(dst_hbm.at[pl.ds(wid * CPT, CPT)], dst_v)

    def _zacc(j, _):
        pltpu.sync_copy(
            rows_v, psum_acc_ref.at[pl.ds(s * ROWS_PER_TILE + j * CHUNK, CHUNK)])
        return 0

    lax.fori_loop(0, ROWS_PER_TILE // CHUNK, _zacc, 0)
    plsc.subcore_barrier()

    ones16 = jnp.ones((16,), jnp.float32)

    def _edge_chunk(j, _):
        pltpu.async_copy(y_hbm.at[src_v.at[j]], rows_v, sem).wait()
        pltpu.sync_copy(rows_v, psum_acc_ref.at[dst_v.at[j]], add=True)
        if with_deg:
            for l in range(CHUNK // 16):
                idx = dst_v[j, pl.ds(l * 16, 16)]
                plsc.addupdate_scatter(deg_v, [idx], ones16)
        return 0

    lax.fori_loop(0, CPT, _edge_chunk, 0)
    plsc.subcore_barrier()

    pltpu.sync_copy(
        psum_acc_ref.at[pl.ds(s * ROWS_PER_TILE, ROWS_PER_TILE)],
        psum_hbm.at[c].at[pl.ds(s * ROWS_PER_TILE, ROWS_PER_TILE)],
    )
    if with_deg:
        pltpu.sync_copy(deg_v, deg_hbm.at[pl.ds(wid * N_ACC, N_ACC)])


def _make_sc_agg(with_deg):
    mesh = plsc.VectorSubcoreMesh(core_axis_name="c", subcore_axis_name="s")
    out_type = [jax.ShapeDtypeStruct((NC, N_ACC, D), jnp.float32)]
    if with_deg:
        out_type.append(jax.ShapeDtypeStruct((NT * N_ACC,), jnp.float32))
    scratch = [
        pltpu.VMEM((CPT, CHUNK), jnp.int32),
        pltpu.VMEM((CPT, CHUNK), jnp.int32),
        pltpu.VMEM((CHUNK, D), jnp.float32),
        pltpu.VMEM((N_ACC,), jnp.float32),
        pltpu.VMEM_SHARED((N_ACC, D), jnp.float32),
        pltpu.SemaphoreType.DMA,
    ]
    return pl.kernel(
        functools.partial(_sc_agg_body, with_deg),
        out_type=out_type,
        mesh=mesh,
        scratch_types=scratch,
        compiler_params=pltpu.CompilerParams(needs_layout_passes=False),
        name="sc_segment_sum" + ("_deg" if with_deg else ""),
    )


_sc_agg_deg = _make_sc_agg(True)
_sc_agg = _make_sc_agg(False)

BR = 1000


def _mm_bias_body(x_ref, w_ref, b_ref, o_ref):
    o_ref[...] = (
        lax.dot_general(x_ref[...], w_ref[...], (((1,), (1,)), ((), ())),
                        preferred_element_type=jnp.float32)
        + b_ref[...]
    )


def _mm_bias(x, w, b):
    return pl.pallas_call(
        _mm_bias_body,
        grid=(N // BR,),
        in_specs=[
            pl.BlockSpec((BR, D), lambda i: (i, 0)),
            pl.BlockSpec((D, D), lambda i: (0, 0)),
            pl.BlockSpec((1, D), lambda i: (0, 0)),
        ],
        out_specs=pl.BlockSpec((BR, D), lambda i: (i, 0)),
        out_shape=jax.ShapeDtypeStruct((N, D), jnp.float32),
    )(x, w, b.reshape(1, D))


def _combine_body(final, psum_ref, deg_ref, z_ref, w_ref, o_ref):
    p = psum_ref[0] + psum_ref[1]
    deg = jnp.sum(deg_ref[...], axis=0)
    recip = 1.0 / jnp.maximum(deg, 1.0)
    agg = p * recip
    a = lax.dot_general(agg, w_ref[...], (((1,), (1,)), ((), ())),
                        preferred_element_type=jnp.float32) + z_ref[...]
    if final:
        m = jnp.max(a, axis=1, keepdims=True)
        lse = jnp.log(jnp.sum(jnp.exp(a - m), axis=1, keepdims=True)) + m
        o_ref[...] = a - lse
    else:
        o_ref[...] = jnp.maximum(a, 0.0)


def _combine(final, psum, degp, z, w):
    return pl.pallas_call(
        functools.partial(_combine_body, final),
        grid=(N // BR,),
        in_specs=[
            pl.BlockSpec((NC, BR, D), lambda i: (0, i, 0)),
            pl.BlockSpec((NT, BR, 1), lambda i: (0, i, 0)),
            pl.BlockSpec((BR, D), lambda i: (i, 0)),
            pl.BlockSpec((D, D), lambda i: (0, 0)),
        ],
        out_specs=pl.BlockSpec((BR, D), lambda i: (i, 0)),
        out_shape=jax.ShapeDtypeStruct((N, D), jnp.float32),
    )(psum, degp, z, w)


def kernel(x, edge_index, W_l0, b_l0, W_r0, W_l1, b_l1, W_r1):
    src = edge_index[0].astype(jnp.int32)
    dst = edge_index[1].astype(jnp.int32)
    pad = E_PAD - E
    src_p = jnp.concatenate([src, jnp.zeros((pad,), jnp.int32)]).reshape(NT * CPT, CHUNK)
    dst_p = jnp.concatenate([dst, jnp.full((pad,), N, jnp.int32)]).reshape(NT * CPT, CHUNK)

    psum0, degp = _sc_agg_deg(x, src_p, dst_p)
    degp = degp.reshape(NT, N_ACC)[:, :N].reshape(NT, N, 1)
    z0 = _mm_bias(x, W_r0, b_l0)
    h1 = _combine(False, psum0[:, :N, :], degp, z0, W_l0)

    (psum1,) = _sc_agg(h1, src_p, dst_p)
    z1 = _mm_bias(h1, W_r1, b_l1)
    return _combine(True, psum1[:, :N, :], degp, z1, W_l1)


# speedup vs baseline: 3.4339x; 1.1779x over previous
import functools

import jax
import jax.numpy as jnp
from jax import lax
from jax.experimental import pallas as pl
from jax.experimental.pallas import tpu as pltpu
from jax.experimental.pallas import tpu_sc as plsc

N = 10000
E = 320000
D = 128

NC = 2
NS = 16
NT = NC * NS
CHUNK = 128
CPT = 80
E_PAD = NT * CPT * CHUNK
N_ACC = 10240

ROWS_PER_TILE = N_ACC // NS

IB = 8
KB = CPT // IB


def _sc_agg_body(y_hbm, src_hbm, dst_hbm, psum_hbm,
                 src_v, dstb0, dstb1, buf0, buf1, acc,
                 sem_g0, sem_g1, sem_s0, sem_s1, sem_i0, sem_i1):
    bufs = (buf0, buf1)
    dstb = (dstb0, dstb1)
    sem_g = (sem_g0, sem_g1)
    sem_s = (sem_s0, sem_s1)
    sem_i = (sem_i0, sem_i1)
    c = lax.axis_index("c")
    s = lax.axis_index("s")
    wid = c * NS + s
    zero16 = jnp.zeros((16,), jnp.float32)

    def _zrow(r, _):
        for l in range(D // 16):
            buf0[r, pl.ds(l * 16, 16)] = zero16
        return 0
    lax.fori_loop(0, CHUNK, _zrow, 0)

    pltpu.sync_copy(src_hbm.at[pl.ds(wid * CPT, CPT)], src_v)
    pltpu.sync_copy(dst_hbm.at[pl.ds(wid * CPT, IB)], dstb0)

    def _zacc(j, _):
        pltpu.sync_copy(
            buf0, acc.at[pl.ds(s * ROWS_PER_TILE + j * CHUNK, CHUNK)])
        return 0

    lax.fori_loop(0, ROWS_PER_TILE // CHUNK, _zacc, 0)

    def _gather_start(j, b):
        pltpu.async_copy(y_hbm.at[src_v.at[j]], bufs[b], sem_g[b])

    def _gather_wait(j, b):
        pltpu.make_async_copy(y_hbm.at[src_v.at[j]], bufs[b], sem_g[b]).wait()

    def _scatter_wait(b):
        pltpu.make_async_copy(bufs[b], acc.at[dstb0.at[0]], sem_s[b]).wait()

    _gather_start(0, 0)
    plsc.subcore_barrier()

    def _block(B, pb):
        @pl.when(B + 1 < KB)
        def _():
            pltpu.async_copy(
                dst_hbm.at[pl.ds(wid * CPT + (B + 1) * IB, IB)],
                dstb[1 - pb], sem_i[1 - pb])

        @pl.when(B >= 1)
        def _():
            pltpu.make_async_copy(
                dst_hbm.at[pl.ds(wid * CPT, IB)], dstb[pb], sem_i[pb]).wait()

        for r in range(IB):
            j = B * IB + r
            b = r % 2
            _gather_wait(j, b)

            @pl.when(j + 1 < CPT)
            def _():
                @pl.when(j >= 1)
                def _():
                    _scatter_wait(1 - b)
                _gather_start(j + 1, 1 - b)

            pltpu.async_copy(bufs[b], acc.at[dstb[pb].at[r]], sem_s[b],
                             add=True)

    def _super(t, _):
        _block(2 * t, 0)
        _block(2 * t + 1, 1)
        return 0

    lax.fori_loop(0, KB // 2, _super, 0)

    _scatter_wait((CPT - 1) % 2)
    plsc.subcore_barrier()

    pltpu.sync_copy(
        acc.at[pl.ds(s * ROWS_PER_TILE, ROWS_PER_TILE)],
        psum_hbm.at[c].at[pl.ds(s * ROWS_PER_TILE, ROWS_PER_TILE)],
    )


_sc_agg = pl.kernel(
    _sc_agg_body,
    out_type=[jax.ShapeDtypeStruct((NC, N_ACC, D), jnp.float32)],
    mesh=plsc.VectorSubcoreMesh(core_axis_name="c", subcore_axis_name="s"),
    scratch_types=[
        pltpu.VMEM((CPT, CHUNK), jnp.int32),
        pltpu.VMEM((IB, CHUNK), jnp.int32),
        pltpu.VMEM((IB, CHUNK), jnp.int32),
        pltpu.VMEM((CHUNK, D), jnp.float32),
        pltpu.VMEM((CHUNK, D), jnp.float32),
        pltpu.VMEM_SHARED((N_ACC, D), jnp.float32),
    ] + [pltpu.SemaphoreType.DMA] * 6,
    compiler_params=pltpu.CompilerParams(needs_layout_passes=False),
    name="sc_segment_sum",
)


def _sc_deg_body(dst_hbm, deg_hbm, dst_v, deg_v):
    c = lax.axis_index("c")
    s = lax.axis_index("s")
    wid = c * NS + s
    zero16 = jnp.zeros((16,), jnp.float32)
    ones16 = jnp.ones((16,), jnp.float32)

    pltpu.sync_copy(dst_hbm.at[pl.ds(wid * CPT, CPT)], dst_v)

    def _zdeg(i, _):
        deg_v[pl.ds(i * 16, 16)] = zero16
        return 0
    lax.fori_loop(0, N_ACC // 16, _zdeg, 0)

    def _chunk(j, _):
        for l in range(CHUNK // 16):
            idx = dst_v[j, pl.ds(l * 16, 16)]
            plsc.addupdate_scatter(deg_v, [idx], ones16)
        return 0
    lax.fori_loop(0, CPT, _chunk, 0)

    pltpu.sync_copy(deg_v, deg_hbm.at[pl.ds(wid * N_ACC, N_ACC)])


_sc_deg = pl.kernel(
    _sc_deg_body,
    out_type=[jax.ShapeDtypeStruct((NT * N_ACC,), jnp.float32)],
    mesh=plsc.VectorSubcoreMesh(core_axis_name="c", subcore_axis_name="s"),
    scratch_types=[
        pltpu.VMEM((CPT, CHUNK), jnp.int32),
        pltpu.VMEM((N_ACC,), jnp.float32),
    ],
    compiler_params=pltpu.CompilerParams(needs_layout_passes=False),
    name="sc_degree",
)

BR = 1000


def _mm_bias_body(x_ref, w_ref, b_ref, o_ref):
    o_ref[...] = (
        lax.dot_general(x_ref[...], w_ref[...], (((1,), (1,)), ((), ())),
                        preferred_element_type=jnp.float32)
        + b_ref[...]
    )


def _mm_bias(x, w, b):
    return pl.pallas_call(
        _mm_bias_body,
        grid=(N // BR,),
        in_specs=[
            pl.BlockSpec((BR, D), lambda i: (i, 0)),
            pl.BlockSpec((D, D), lambda i: (0, 0)),
            pl.BlockSpec((1, D), lambda i: (0, 0)),
        ],
        out_specs=pl.BlockSpec((BR, D), lambda i: (i, 0)),
        out_shape=jax.ShapeDtypeStruct((N, D), jnp.float32),
    )(x, w, b.reshape(1, D))


def _combine_body(final, psum_ref, deg_ref, z_ref, w_ref, o_ref):
    p = psum_ref[0] + psum_ref[1]
    deg = jnp.sum(deg_ref[...], axis=0)
    recip = 1.0 / jnp.maximum(deg, 1.0)
    agg = p * recip
    a = lax.dot_general(agg, w_ref[...], (((1,), (1,)), ((), ())),
                        preferred_element_type=jnp.float32) + z_ref[...]
    if final:
        m = jnp.max(a, axis=1, keepdims=True)
        lse = jnp.log(jnp.sum(jnp.exp(a - m), axis=1, keepdims=True)) + m
        o_ref[...] = a - lse
    else:
        o_ref[...] = jnp.maximum(a, 0.0)


def _combine(final, psum, degp, z, w):
    return pl.pallas_call(
        functools.partial(_combine_body, final),
        grid=(N // BR,),
        in_specs=[
            pl.BlockSpec((NC, BR, D), lambda i: (0, i, 0)),
            pl.BlockSpec((NT, BR, 1), lambda i: (0, i, 0)),
            pl.BlockSpec((BR, D), lambda i: (i, 0)),
            pl.BlockSpec((D, D), lambda i: (0, 0)),
        ],
        out_specs=pl.BlockSpec((BR, D), lambda i: (i, 0)),
        out_shape=jax.ShapeDtypeStruct((N, D), jnp.float32),
    )(psum, degp, z, w)


def kernel(x, edge_index, W_l0, b_l0, W_r0, W_l1, b_l1, W_r1):
    src = edge_index[0].astype(jnp.int32)
    dst = edge_index[1].astype(jnp.int32)
    pad = E_PAD - E
    src_p = jnp.concatenate([src, jnp.zeros((pad,), jnp.int32)]).reshape(NT * CPT, CHUNK)
    dst_p = jnp.concatenate([dst, jnp.full((pad,), N, jnp.int32)]).reshape(NT * CPT, CHUNK)

    (degp,) = _sc_deg(dst_p)
    degp = degp.reshape(NT, N_ACC)[:, :N].reshape(NT, N, 1)
    (psum0,) = _sc_agg(x, src_p, dst_p)
    z0 = _mm_bias(x, W_r0, b_l0)
    h1 = _combine(False, psum0[:, :N, :], degp, z0, W_l0)

    (psum1,) = _sc_agg(h1, src_p, dst_p)
    z1 = _mm_bias(h1, W_r1, b_l1)
    return _combine(True, psum1[:, :N, :], degp, z1, W_l1)


# speedup vs baseline: 4.5375x; 1.3214x over previous
import functools

import jax
import jax.numpy as jnp
from jax import lax
from jax.experimental import pallas as pl
from jax.experimental.pallas import tpu as pltpu
from jax.experimental.pallas import tpu_sc as plsc

N = 10000
E = 320000
D = 128
DH = D // 2

NC = 2
NS = 16
NT = NC * NS
CHUNK = 128
CPT = 160
E_PAD = NS * CPT * CHUNK
NP = 10240

ROWS_PER_TILE = NP // NS

IB = 8
KB = CPT // IB

DCPT = E_PAD // (NT * CHUNK)


def _sc_agg_body(yy_hbm, src_hbm, dst_hbm, psum_hbm,
                 srcb0, srcb1, dstb0, dstb1, buf0, buf1, acc,
                 sem_g0, sem_g1, sem_s0, sem_s1,
                 sem_is0, sem_is1, sem_id0, sem_id1):
    bufs = (buf0, buf1)
    srcb = (srcb0, srcb1)
    dstb = (dstb0, dstb1)
    sem_g = (sem_g0, sem_g1)
    sem_s = (sem_s0, sem_s1)
    sem_is = (sem_is0, sem_is1)
    sem_id = (sem_id0, sem_id1)
    c = lax.axis_index("c")
    s = lax.axis_index("s")
    y_hbm = yy_hbm.at[c]
    zero16 = jnp.zeros((16,), jnp.float32)

    def _zrow(r, _):
        for l in range(DH // 16):
            buf0[r, pl.ds(l * 16, 16)] = zero16
        return 0
    lax.fori_loop(0, CHUNK, _zrow, 0)

    pltpu.sync_copy(src_hbm.at[pl.ds(s * CPT, IB)], srcb0)
    pltpu.sync_copy(dst_hbm.at[pl.ds(s * CPT, IB)], dstb0)

    rows = pl.ds(s * ROWS_PER_TILE, ROWS_PER_TILE)

    def _zacc(j, _):
        pltpu.sync_copy(
            buf0, acc.at[pl.ds(s * ROWS_PER_TILE + j * CHUNK, CHUNK)])
        return 0

    lax.fori_loop(0, ROWS_PER_TILE // CHUNK, _zacc, 0)

    def _gather_start(q, r, b):
        pltpu.async_copy(y_hbm.at[srcb[q].at[r]], bufs[b], sem_g[b])

    def _gather_wait(q, r, b):
        pltpu.make_async_copy(y_hbm.at[srcb[q].at[r]], bufs[b], sem_g[b]).wait()

    def _scatter_wait(b):
        pltpu.make_async_copy(bufs[b], acc.at[dstb0.at[0]], sem_s[b]).wait()

    plsc.subcore_barrier()
    _gather_start(0, 0, 0)

    def _block(B, pb):
        qb = 1 - pb
        @pl.when(B >= 1)
        def _():
            pltpu.make_async_copy(
                dst_hbm.at[pl.ds(s * CPT, IB)], dstb[pb], sem_id[pb]).wait()

        for r in range(IB):
            b = r % 2
            _gather_wait(pb, r, b)

            if r + 1 < IB:
                @pl.when(B * IB + r >= 1)
                def _():
                    _scatter_wait(1 - b)
                _gather_start(pb, r + 1, 1 - b)
            else:
                @pl.when(B + 1 < KB)
                def _():
                    _scatter_wait(1 - b)
                    pltpu.make_async_copy(
                        src_hbm.at[pl.ds(s * CPT, IB)], srcb[qb],
                        sem_is[qb]).wait()
                    _gather_start(qb, 0, 1 - b)

            pltpu.async_copy(bufs[b], acc.at[dstb[pb].at[r]], sem_s[b],
                             add=True)

            if r == 0:
                @pl.when(B + 1 < KB)
                def _():
                    pltpu.async_copy(
                        src_hbm.at[pl.ds(s * CPT + (B + 1) * IB, IB)],
                        srcb[qb], sem_is[qb])
                    pltpu.async_copy(
                        dst_hbm.at[pl.ds(s * CPT + (B + 1) * IB, IB)],
                        dstb[qb], sem_id[qb])

    def _super(t, _):
        _block(2 * t, 0)
        _block(2 * t + 1, 1)
        return 0

    lax.fori_loop(0, KB // 2, _super, 0)

    _scatter_wait(0)
    _scatter_wait(1)
    plsc.subcore_barrier()

    pltpu.sync_copy(acc.at[rows], psum_hbm.at[c].at[rows])


_sc_agg = pl.kernel(
    _sc_agg_body,
    out_type=[jax.ShapeDtypeStruct((NC, NP, DH), jnp.float32)],
    mesh=plsc.VectorSubcoreMesh(core_axis_name="c", subcore_axis_name="s"),
    scratch_types=[
        pltpu.VMEM((IB, CHUNK), jnp.int32),
        pltpu.VMEM((IB, CHUNK), jnp.int32),
        pltpu.VMEM((IB, CHUNK), jnp.int32),
        pltpu.VMEM((IB, CHUNK), jnp.int32),
        pltpu.VMEM((CHUNK, DH), jnp.float32),
        pltpu.VMEM((CHUNK, DH), jnp.float32),
        pltpu.VMEM_SHARED((NP, DH), jnp.float32),
    ] + [pltpu.SemaphoreType.DMA] * 8,
    compiler_params=pltpu.CompilerParams(needs_layout_passes=False,
                                         use_tc_tiling_on_sc=False),
    name="sc_segment_sum",
)


def _sc_deg_body(dst_hbm, deg_hbm, dst_v, deg_v, degblk_v, degsum_v, deg_sh):
    c = lax.axis_index("c")
    s = lax.axis_index("s")
    wid = c * NS + s
    zero16 = jnp.zeros((16,), jnp.float32)
    ones16 = jnp.ones((16,), jnp.float32)

    pltpu.sync_copy(dst_hbm.at[pl.ds(wid * DCPT, DCPT)], dst_v)

    def _zdeg(i, _):
        deg_v[pl.ds(i * 16, 16)] = zero16
        return 0
    lax.fori_loop(0, NP // 16, _zdeg, 0)

    def _chunk(j, _):
        for l in range(CHUNK // 16):
            idx = dst_v[j, pl.ds(l * 16, 16)]
            plsc.addupdate_scatter(deg_v, [idx], ones16)
        return 0
    lax.fori_loop(0, DCPT, _chunk, 0)

    pltpu.sync_copy(deg_v, deg_sh.at[s])
    plsc.subcore_barrier()
    pltpu.sync_copy(deg_sh.at[:, pl.ds(s * ROWS_PER_TILE, ROWS_PER_TILE)],
                    degblk_v)

    def _red(o, _):
        tot = degblk_v[0, pl.ds(o * 16, 16)]
        for r in range(1, NS):
            tot = tot + degblk_v[r, pl.ds(o * 16, 16)]
        degsum_v[pl.ds(o * 16, 16)] = tot
        return 0
    lax.fori_loop(0, ROWS_PER_TILE // 16, _red, 0)

    pltpu.sync_copy(
        degsum_v, deg_hbm.at[pl.ds(c * NP + s * ROWS_PER_TILE, ROWS_PER_TILE)])


_sc_deg = pl.kernel(
    _sc_deg_body,
    out_type=[jax.ShapeDtypeStruct((NC * NP,), jnp.float32)],
    mesh=plsc.VectorSubcoreMesh(core_axis_name="c", subcore_axis_name="s"),
    scratch_types=[
        pltpu.VMEM((DCPT, CHUNK), jnp.int32),
        pltpu.VMEM((NP,), jnp.float32),
        pltpu.VMEM((NS, ROWS_PER_TILE), jnp.float32),
        pltpu.VMEM((ROWS_PER_TILE,), jnp.float32),
        pltpu.VMEM_SHARED((NS, NP), jnp.float32),
    ],
    compiler_params=pltpu.CompilerParams(needs_layout_passes=False),
    name="sc_degree",
)

BR = 1024


def _mm_bias_body(x_ref, w_ref, b_ref, o_ref):
    x = jnp.concatenate([x_ref[0], x_ref[1]], axis=1)
    o_ref[...] = (
        lax.dot_general(x, w_ref[...], (((1,), (1,)), ((), ())),
                        preferred_element_type=jnp.float32)
        + b_ref[...]
    )


def _mm_bias(xh, w, b):
    return pl.pallas_call(
        _mm_bias_body,
        grid=(NP // BR,),
        in_specs=[
            pl.BlockSpec((NC, BR, DH), lambda i: (0, i, 0)),
            pl.BlockSpec((D, D), lambda i: (0, 0)),
            pl.BlockSpec((1, D), lambda i: (0, 0)),
        ],
        out_specs=pl.BlockSpec((BR, D), lambda i: (i, 0)),
        out_shape=jax.ShapeDtypeStruct((NP, D), jnp.float32),
    )(xh, w, b.reshape(1, D))


def _combine_body(final, psum_ref, deg_ref, z_ref, w_ref, o_ref):
    p = jnp.concatenate([psum_ref[0], psum_ref[1]], axis=1)
    recip = 1.0 / jnp.maximum(deg_ref[...], 1.0)
    agg = p * recip
    a = lax.dot_general(agg, w_ref[...], (((1,), (1,)), ((), ())),
                        preferred_element_type=jnp.float32) + z_ref[...]
    if final:
        m = jnp.max(a, axis=1, keepdims=True)
        lse = jnp.log(jnp.sum(jnp.exp(a - m), axis=1, keepdims=True)) + m
        o_ref[...] = a - lse
    else:
        h = jnp.maximum(a, 0.0)
        o_ref[0] = h[:, :DH]
        o_ref[1] = h[:, DH:]


def _combine(final, psum, deg, z, w):
    if final:
        out_specs = pl.BlockSpec((BR, D), lambda i: (i, 0))
        out_shape = jax.ShapeDtypeStruct((NP, D), jnp.float32)
    else:
        out_specs = pl.BlockSpec((NC, BR, DH), lambda i: (0, i, 0))
        out_shape = jax.ShapeDtypeStruct((NC, NP, DH), jnp.float32)
    return pl.pallas_call(
        functools.partial(_combine_body, final),
        grid=(NP // BR,),
        in_specs=[
            pl.BlockSpec((NC, BR, DH), lambda i: (0, i, 0)),
            pl.BlockSpec((BR, 1), lambda i: (i, 0)),
            pl.BlockSpec((BR, D), lambda i: (i, 0)),
            pl.BlockSpec((D, D), lambda i: (0, 0)),
        ],
        out_specs=out_specs,
        out_shape=out_shape,
    )(psum, deg, z, w)


def kernel(x, edge_index, W_l0, b_l0, W_r0, W_l1, b_l1, W_r1):
    src = edge_index[0].astype(jnp.int32)
    dst = edge_index[1].astype(jnp.int32)
    pad = E_PAD - E
    src_p = jnp.concatenate([src, jnp.zeros((pad,), jnp.int32)]).reshape(NS * CPT, CHUNK)
    dst_p = jnp.concatenate([dst, jnp.full((pad,), N, jnp.int32)]).reshape(NS * CPT, CHUNK)

    xp = jnp.pad(x, ((0, NP - N), (0, 0)))
    xh = jnp.stack([xp[:, :DH], xp[:, DH:]])

    (degp,) = _sc_deg(dst_p)
    degp = degp.reshape(NC, NP)
    deg = (degp[0] + degp[1]).reshape(NP, 1)

    (psum0,) = _sc_agg(xh, src_p, dst_p)
    z0 = _mm_bias(xh, W_r0, b_l0)
    hh = _combine(False, psum0, deg, z0, W_l0)

    (psum1,) = _sc_agg(hh, src_p, dst_p)
    z1 = _mm_bias(hh, W_r1, b_l1)
    return _combine(True, psum1, deg, z1, W_l1)[:N]
